# baseline (speedup 1.0000x reference)
import functools

import jax
import jax.numpy as jnp
from jax import lax
from jax.experimental import pallas as pl
from jax.experimental.pallas import tpu as pltpu
from jax.experimental.pallas import tpu_sc as plsc

B, T, Y = 16, 512, 2048
DX = 512
SPK = 64
ALIGN = 512
FE = 64
NSPK = 128
OUT_W = ALIGN + FE + FE + 1
ZROW = B * T
TBL_ROWS = B * T + T
NC, NS = 2, 16
NW = NC * NS
RPW = (B * Y) // NW
CH = 128
NCH = RPW // CH


def _tc_table_body(x_ref, wlin_ref, blin_ref, emb_ref, spk_ref, dur_ref,
                   tbl_ref, cse_ref):
    pid = pl.program_id(0)
    w1 = wlin_ref[:DX, :]
    w2 = wlin_ref[DX:, :]
    e2 = jnp.dot(emb_ref[...], w2, preferred_element_type=jnp.float32)
    srow = spk_ref[0, 0, :]
    oh = (srow[:, None] ==
          lax.broadcasted_iota(jnp.int32, (T, NSPK), 1)).astype(jnp.float32)
    h = (jnp.dot(x_ref[0], w1, preferred_element_type=jnp.float32)
         + jnp.dot(oh, e2, preferred_element_type=jnp.float32)
         + blin_ref[0, :])
    h = jnp.where(pid < B, h, 0.0)
    tbl_ref[...] = h

    dur_f = dur_ref[0, 0, :].astype(jnp.float32)
    dur2 = dur_f.reshape(4, 128)
    upt = (lax.broadcasted_iota(jnp.int32, (128, 128), 0) <=
           lax.broadcasted_iota(jnp.int32, (128, 128), 1)).astype(jnp.float32)
    cw = jnp.dot(dur2, upt, preferred_element_type=jnp.float32)
    offm = (lax.broadcasted_iota(jnp.int32, (4, 4), 0) >
            lax.broadcasted_iota(jnp.int32, (4, 4), 1)).astype(jnp.float32)
    off = jnp.dot(offm, cw[:, 127:128], preferred_element_type=jnp.float32)
    cse = (cw + off).reshape(T) - dur_f
    cse_ref[0, 0, :] = cse.astype(jnp.int32)


def _tc_table(x, W_lin, b_lin2, emb_speaker, spk3, dur3):
    return pl.pallas_call(
        _tc_table_body,
        grid=(B + 1,),
        in_specs=[
            pl.BlockSpec((1, T, DX), lambda b: (jnp.minimum(b, B - 1), 0, 0)),
            pl.BlockSpec((DX + SPK, ALIGN), lambda b: (0, 0)),
            pl.BlockSpec((1, ALIGN), lambda b: (0, 0)),
            pl.BlockSpec((NSPK, SPK), lambda b: (0, 0)),
            pl.BlockSpec((1, 1, T), lambda b: (jnp.minimum(b, B - 1), 0, 0)),
            pl.BlockSpec((1, 1, T), lambda b: (jnp.minimum(b, B - 1), 0, 0)),
        ],
        out_specs=[
            pl.BlockSpec((T, ALIGN), lambda b: (b, 0)),
            pl.BlockSpec((1, 1, T), lambda b: (jnp.minimum(b, B - 1), 0, 0)),
        ],
        out_shape=[
            jax.ShapeDtypeStruct((TBL_ROWS, ALIGN), jnp.float32),
            jax.ShapeDtypeStruct((B, 1, T), jnp.int32),
        ],
    )(x, W_lin, b_lin2, emb_speaker, spk3, dur3)


def _sc_body(tbl_h, dur_h, cse_h, f0_h, rmse_h, pos_h, wf0_h, bf0_h, wrm_h,
             brm_h, out_h, durv, csev, fidxv, f0v, rmv, posv, wf0v, bf0v,
             wrmv, brmv, buf, sem):
    cid = lax.axis_index("c")
    sid = lax.axis_index("s")
    wid = sid * NC + cid
    bb = wid // 2
    half = wid % 2
    base = wid * RPW

    pltpu.sync_copy(dur_h.at[pl.ds(bb * T, T)], durv)
    pltpu.sync_copy(cse_h.at[pl.ds(bb * T, T)], csev)
    pltpu.sync_copy(f0_h.at[pl.ds(base, RPW)], f0v)
    pltpu.sync_copy(rmse_h.at[pl.ds(base, RPW)], rmv)
    pltpu.sync_copy(pos_h.at[pl.ds(base, RPW)], posv)
    pltpu.sync_copy(wf0_h, wf0v)
    pltpu.sync_copy(bf0_h, bf0v)
    pltpu.sync_copy(wrm_h, wrmv)
    pltpu.sync_copy(brm_h, brmv)

    zlane = jnp.full((16,), ZROW, jnp.int32) + lax.iota(jnp.int32, 16)

    def fill(i, c):
        fidxv[pl.ds(i * 16, 16)] = zlane + lax.rem(i * 16, T)
        return c

    lax.fori_loop(0, Y // 16, fill, 0)

    rowbase = bb * T
    for v in range(T // 16):
        dv = durv[pl.ds(v * 16, 16)]
        pos = csev[pl.ds(v * 16, 16)]
        vals = (jnp.full((16,), rowbase + v * 16, jnp.int32)
                + lax.iota(jnp.int32, 16))
        plsc.store_scatter(fidxv, [pos], vals, mask=dv >= 1)
        plsc.store_scatter(fidxv, [pos + 1], vals, mask=dv >= 2)
        plsc.store_scatter(fidxv, [pos + 2], vals, mask=dv >= 3)

    nj = FE // 16
    wf0r = [wf0v[pl.ds(j * 16, 16)] for j in range(nj)]
    bf0r = [bf0v[pl.ds(j * 16, 16)] for j in range(nj)]
    wrmr = [wrmv[pl.ds(j * 16, 16)] for j in range(nj)]
    brmr = [brmv[pl.ds(j * 16, 16)] for j in range(nj)]
    lane = lax.iota(jnp.int32, 16)
    poscol = jnp.full((16,), ALIGN + 2 * FE, jnp.int32)

    foff = half * (Y // 2)
    dtail = durv[pl.ds(T - 16, 16)]
    ctail = csev[pl.ds(T - 16, 16)]
    total = dtail[15] + ctail[15]
    zvec = jnp.zeros((16,), jnp.float32)

    def chunk_fn(c, acc):
        start = foff + c * CH

        @pl.when(total > start)
        def _():
            idx_slice = fidxv.at[pl.ds(start, CH)]
            pltpu.async_copy(tbl_h.at[idx_slice], buf.at[:, pl.ds(0, ALIGN)],
                             sem).wait()

        @pl.when(jnp.logical_and(
            total <= start,
            jnp.logical_or(c == 0, total > start - CH)))
        def _():
            def zrow_fn(r, acc2):
                for j in range(ALIGN // 16):
                    buf[r, pl.ds(j * 16, 16)] = zvec
                return acc2

            lax.fori_loop(0, CH, zrow_fn, 0)

        def grp_fn(g, acc2):
            rbase = c * CH + g * 16
            f0g = f0v[pl.ds(rbase, 16)]
            rmg = rmv[pl.ds(rbase, 16)]
            pog = posv[pl.ds(rbase, 16)]
            plsc.store_scatter(buf, [g * 16 + lane, poscol], pog)
            for r in range(16):
                row = g * 16 + r
                f0r = f0g[r]
                rmr = rmg[r]
                for j in range(nj):
                    buf[row, pl.ds(ALIGN + j * 16, 16)] = (
                        f0r * wf0r[j] + bf0r[j])
                    buf[row, pl.ds(ALIGN + FE + j * 16, 16)] = (
                        rmr * wrmr[j] + brmr[j])
            return acc2

        lax.fori_loop(0, CH // 16, grp_fn, 0)
        pltpu.sync_copy(buf, out_h.at[bb, pl.ds(start, CH)])
        return acc

    lax.fori_loop(0, NCH, chunk_fn, 0)


def _sc_expand(tbl, dur_flat, cse_flat, f0_flat, rmse_flat, pos_flat,
               wf0, bf0, wrm, brm):
    mesh = plsc.VectorSubcoreMesh(core_axis_name="c", subcore_axis_name="s")
    fn = functools.partial(
        pl.kernel,
        mesh=mesh,
        compiler_params=pltpu.CompilerParams(needs_layout_passes=False),
        out_type=jax.ShapeDtypeStruct((B, Y, OUT_W), jnp.float32),
        scratch_types=[
            pltpu.VMEM((T,), jnp.int32),
            pltpu.VMEM((T,), jnp.int32),
            pltpu.VMEM((Y,), jnp.int32),
            pltpu.VMEM((RPW,), jnp.float32),
            pltpu.VMEM((RPW,), jnp.float32),
            pltpu.VMEM((RPW,), jnp.float32),
            pltpu.VMEM((FE,), jnp.float32),
            pltpu.VMEM((FE,), jnp.float32),
            pltpu.VMEM((FE,), jnp.float32),
            pltpu.VMEM((FE,), jnp.float32),
            pltpu.VMEM((CH, OUT_W), jnp.float32),
            pltpu.SemaphoreType.DMA,
        ],
    )(_sc_body)
    return fn(tbl, dur_flat, cse_flat, f0_flat, rmse_flat, pos_flat,
              wf0, bf0, wrm, brm)


def kernel(x, f0, rmse, position, emb_speaker, W_f0, b_f0, W_rmse, b_rmse,
           W_lin, b_lin, speaker, duration, max_y_len):
    del max_y_len
    tbl, cse = _tc_table(x, W_lin, b_lin.reshape(1, ALIGN), emb_speaker,
                         speaker.reshape(B, 1, T), duration.reshape(B, 1, T))
    out = _sc_expand(tbl,
                     duration.reshape(-1), cse.reshape(-1),
                     f0.reshape(-1), rmse.reshape(-1), position.reshape(-1),
                     W_f0.reshape(-1), b_f0,
                     W_rmse.reshape(-1), b_rmse)
    return out


# speedup vs baseline: 1.0056x; 1.0056x over previous
import functools

import jax
import jax.numpy as jnp
from jax import lax
from jax.experimental import pallas as pl
from jax.experimental.pallas import tpu as pltpu
from jax.experimental.pallas import tpu_sc as plsc

B, T, Y = 16, 512, 2048
DX = 512
SPK = 64
ALIGN = 512
FE = 64
NSPK = 128
OUT_W = ALIGN + FE + FE + 1
ZROW = B * T
TBL_ROWS = B * T + T
NC, NS = 2, 16
NW = NC * NS
RPW = (B * Y) // NW
CH = 128
NCH = RPW // CH


def _tc_table_body(x_ref, wlin_ref, blin_ref, emb_ref, spk_ref, dur_ref,
                   tbl_ref, cse_ref):
    pid = pl.program_id(0)
    w1 = wlin_ref[:DX, :]
    w2 = wlin_ref[DX:, :]
    e2 = jnp.dot(emb_ref[...], w2, preferred_element_type=jnp.float32)
    srow = spk_ref[0, 0, :]
    oh = (srow[:, None] ==
          lax.broadcasted_iota(jnp.int32, (T, NSPK), 1)).astype(jnp.float32)
    h = (jnp.dot(x_ref[0], w1, preferred_element_type=jnp.float32)
         + jnp.dot(oh, e2, preferred_element_type=jnp.float32)
         + blin_ref[0, :])
    h = jnp.where(pid < B, h, 0.0)
    tbl_ref[...] = h

    dur_f = dur_ref[0, 0, :].astype(jnp.float32)
    dur2 = dur_f.reshape(4, 128)
    upt = (lax.broadcasted_iota(jnp.int32, (128, 128), 0) <=
           lax.broadcasted_iota(jnp.int32, (128, 128), 1)).astype(jnp.float32)
    cw = jnp.dot(dur2, upt, preferred_element_type=jnp.float32)
    offm = (lax.broadcasted_iota(jnp.int32, (4, 4), 0) >
            lax.broadcasted_iota(jnp.int32, (4, 4), 1)).astype(jnp.float32)
    off = jnp.dot(offm, cw[:, 127:128], preferred_element_type=jnp.float32)
    cse = (cw + off).reshape(T) - dur_f
    cse_ref[0, 0, :] = cse.astype(jnp.int32)


def _tc_table(x, W_lin, b_lin2, emb_speaker, spk3, dur3):
    return pl.pallas_call(
        _tc_table_body,
        grid=(B + 1,),
        in_specs=[
            pl.BlockSpec((1, T, DX), lambda b: (jnp.minimum(b, B - 1), 0, 0)),
            pl.BlockSpec((DX + SPK, ALIGN), lambda b: (0, 0)),
            pl.BlockSpec((1, ALIGN), lambda b: (0, 0)),
            pl.BlockSpec((NSPK, SPK), lambda b: (0, 0)),
            pl.BlockSpec((1, 1, T), lambda b: (jnp.minimum(b, B - 1), 0, 0)),
            pl.BlockSpec((1, 1, T), lambda b: (jnp.minimum(b, B - 1), 0, 0)),
        ],
        out_specs=[
            pl.BlockSpec((T, ALIGN), lambda b: (b, 0)),
            pl.BlockSpec((1, 1, T), lambda b: (jnp.minimum(b, B - 1), 0, 0)),
        ],
        out_shape=[
            jax.ShapeDtypeStruct((TBL_ROWS, ALIGN), jnp.float32),
            jax.ShapeDtypeStruct((B, 1, T), jnp.int32),
        ],
    )(x, W_lin, b_lin2, emb_speaker, spk3, dur3)


def _sc_body(tbl_h, dur_h, cse_h, f0_h, rmse_h, pos_h, wf0_h, bf0_h, wrm_h,
             brm_h, out_h, durv, csev, fidxv, f0v, rmv, posv, wf0v, bf0v,
             wrmv, brmv, buf, sem):
    cid = lax.axis_index("c")
    sid = lax.axis_index("s")
    wid = sid * NC + cid
    bb = wid // 2
    half = lax.rem(wid + bb, 2)
    base = wid * RPW

    pltpu.sync_copy(dur_h.at[pl.ds(bb * T, T)], durv)
    pltpu.sync_copy(cse_h.at[pl.ds(bb * T, T)], csev)
    pltpu.sync_copy(f0_h.at[pl.ds(base, RPW)], f0v)
    pltpu.sync_copy(rmse_h.at[pl.ds(base, RPW)], rmv)
    pltpu.sync_copy(pos_h.at[pl.ds(base, RPW)], posv)
    pltpu.sync_copy(wf0_h, wf0v)
    pltpu.sync_copy(bf0_h, bf0v)
    pltpu.sync_copy(wrm_h, wrmv)
    pltpu.sync_copy(brm_h, brmv)

    zlane = jnp.full((16,), ZROW, jnp.int32) + lax.iota(jnp.int32, 16)

    def fill(i, c):
        fidxv[pl.ds(i * 16, 16)] = zlane + lax.rem(i * 16, T)
        return c

    lax.fori_loop(0, Y // 16, fill, 0)

    rowbase = bb * T
    for v in range(T // 16):
        dv = durv[pl.ds(v * 16, 16)]
        pos = csev[pl.ds(v * 16, 16)]
        vals = (jnp.full((16,), rowbase + v * 16, jnp.int32)
                + lax.iota(jnp.int32, 16))
        plsc.store_scatter(fidxv, [pos], vals, mask=dv >= 1)
        plsc.store_scatter(fidxv, [pos + 1], vals, mask=dv >= 2)
        plsc.store_scatter(fidxv, [pos + 2], vals, mask=dv >= 3)

    nj = FE // 16
    wf0r = [wf0v[pl.ds(j * 16, 16)] for j in range(nj)]
    bf0r = [bf0v[pl.ds(j * 16, 16)] for j in range(nj)]
    wrmr = [wrmv[pl.ds(j * 16, 16)] for j in range(nj)]
    brmr = [brmv[pl.ds(j * 16, 16)] for j in range(nj)]
    lane = lax.iota(jnp.int32, 16)
    poscol = jnp.full((16,), ALIGN + 2 * FE, jnp.int32)

    foff = half * (Y // 2)
    dtail = durv[pl.ds(T - 16, 16)]
    ctail = csev[pl.ds(T - 16, 16)]
    total = dtail[15] + ctail[15]
    zvec = jnp.zeros((16,), jnp.float32)

    def chunk_fn(c, acc):
        start = foff + c * CH

        @pl.when(total > start)
        def _():
            idx_slice = fidxv.at[pl.ds(start, CH)]
            pltpu.async_copy(tbl_h.at[idx_slice], buf.at[:, pl.ds(0, ALIGN)],
                             sem).wait()

        @pl.when(jnp.logical_and(
            total <= start,
            jnp.logical_or(c == 0, total > start - CH)))
        def _():
            def zrow_fn(r, acc2):
                for j in range(ALIGN // 16):
                    buf[r, pl.ds(j * 16, 16)] = zvec
                return acc2

            lax.fori_loop(0, CH, zrow_fn, 0)

        def grp_fn(g, acc2):
            rbase = c * CH + g * 16
            f0g = f0v[pl.ds(rbase, 16)]
            rmg = rmv[pl.ds(rbase, 16)]
            pog = posv[pl.ds(rbase, 16)]
            plsc.store_scatter(buf, [g * 16 + lane, poscol], pog)
            for r in range(16):
                row = g * 16 + r
                f0r = f0g[r]
                rmr = rmg[r]
                for j in range(nj):
                    buf[row, pl.ds(ALIGN + j * 16, 16)] = (
                        f0r * wf0r[j] + bf0r[j])
                    buf[row, pl.ds(ALIGN + FE + j * 16, 16)] = (
                        rmr * wrmr[j] + brmr[j])
            return acc2

        lax.fori_loop(0, CH // 16, grp_fn, 0)
        pltpu.sync_copy(buf, out_h.at[bb, pl.ds(start, CH)])
        return acc

    lax.fori_loop(0, NCH, chunk_fn, 0)


def _sc_expand(tbl, dur_flat, cse_flat, f0_flat, rmse_flat, pos_flat,
               wf0, bf0, wrm, brm):
    mesh = plsc.VectorSubcoreMesh(core_axis_name="c", subcore_axis_name="s")
    fn = functools.partial(
        pl.kernel,
        mesh=mesh,
        compiler_params=pltpu.CompilerParams(needs_layout_passes=False),
        out_type=jax.ShapeDtypeStruct((B, Y, OUT_W), jnp.float32),
        scratch_types=[
            pltpu.VMEM((T,), jnp.int32),
            pltpu.VMEM((T,), jnp.int32),
            pltpu.VMEM((Y,), jnp.int32),
            pltpu.VMEM((RPW,), jnp.float32),
            pltpu.VMEM((RPW,), jnp.float32),
            pltpu.VMEM((RPW,), jnp.float32),
            pltpu.VMEM((FE,), jnp.float32),
            pltpu.VMEM((FE,), jnp.float32),
            pltpu.VMEM((FE,), jnp.float32),
            pltpu.VMEM((FE,), jnp.float32),
            pltpu.VMEM((CH, OUT_W), jnp.float32),
            pltpu.SemaphoreType.DMA,
        ],
    )(_sc_body)
    return fn(tbl, dur_flat, cse_flat, f0_flat, rmse_flat, pos_flat,
              wf0, bf0, wrm, brm)


def kernel(x, f0, rmse, position, emb_speaker, W_f0, b_f0, W_rmse, b_rmse,
           W_lin, b_lin, speaker, duration, max_y_len):
    del max_y_len
    tbl, cse = _tc_table(x, W_lin, b_lin.reshape(1, ALIGN), emb_speaker,
                         speaker.reshape(B, 1, T), duration.reshape(B, 1, T))
    out = _sc_expand(tbl,
                     duration.reshape(-1), cse.reshape(-1),
                     f0.reshape(-1), rmse.reshape(-1), position.reshape(-1),
                     W_f0.reshape(-1), b_f0,
                     W_rmse.reshape(-1), b_rmse)
    return out


# speedup vs baseline: 1.0064x; 1.0007x over previous
import functools

import jax
import jax.numpy as jnp
from jax import lax
from jax.experimental import pallas as pl
from jax.experimental.pallas import tpu as pltpu
from jax.experimental.pallas import tpu_sc as plsc

B, T, Y = 16, 512, 2048
DX = 512
SPK = 64
ALIGN = 512
FE = 64
NSPK = 128
OUT_W = ALIGN + FE + FE + 1
ZROW = B * T
TBL_ROWS = B * T + T
NC, NS = 2, 16
NW = NC * NS
RPW = (B * Y) // NW
CH = 128
NCH = RPW // CH


def _tc_table_body(x_ref, wlin_ref, blin_ref, emb_ref, spk_ref, dur_ref,
                   tbl_ref, cse_ref):
    pid = pl.program_id(0)
    w1 = wlin_ref[:DX, :]
    w2 = wlin_ref[DX:, :]
    e2 = jnp.dot(emb_ref[...], w2, preferred_element_type=jnp.float32)
    srow = spk_ref[0, 0, :]
    oh = (srow[:, None] ==
          lax.broadcasted_iota(jnp.int32, (T, NSPK), 1)).astype(jnp.float32)
    h = (jnp.dot(x_ref[0], w1, preferred_element_type=jnp.float32)
         + jnp.dot(oh, e2, preferred_element_type=jnp.float32)
         + blin_ref[0, :])
    h = jnp.where(pid < B, h, 0.0)
    tbl_ref[...] = h

    dur_f = dur_ref[0, 0, :].astype(jnp.float32)
    dur2 = dur_f.reshape(4, 128)
    upt = (lax.broadcasted_iota(jnp.int32, (128, 128), 0) <=
           lax.broadcasted_iota(jnp.int32, (128, 128), 1)).astype(jnp.float32)
    cw = jnp.dot(dur2, upt, preferred_element_type=jnp.float32)
    offm = (lax.broadcasted_iota(jnp.int32, (4, 4), 0) >
            lax.broadcasted_iota(jnp.int32, (4, 4), 1)).astype(jnp.float32)
    off = jnp.dot(offm, cw[:, 127:128], preferred_element_type=jnp.float32)
    cse = (cw + off).reshape(T) - dur_f
    cse_ref[0, 0, :] = cse.astype(jnp.int32)


def _tc_table(x, W_lin, b_lin2, emb_speaker, spk3, dur3):
    return pl.pallas_call(
        _tc_table_body,
        grid=(B + 1,),
        in_specs=[
            pl.BlockSpec((1, T, DX), lambda b: (jnp.minimum(b, B - 1), 0, 0)),
            pl.BlockSpec((DX + SPK, ALIGN), lambda b: (0, 0)),
            pl.BlockSpec((1, ALIGN), lambda b: (0, 0)),
            pl.BlockSpec((NSPK, SPK), lambda b: (0, 0)),
            pl.BlockSpec((1, 1, T), lambda b: (jnp.minimum(b, B - 1), 0, 0)),
            pl.BlockSpec((1, 1, T), lambda b: (jnp.minimum(b, B - 1), 0, 0)),
        ],
        out_specs=[
            pl.BlockSpec((T, ALIGN), lambda b: (b, 0)),
            pl.BlockSpec((1, 1, T), lambda b: (jnp.minimum(b, B - 1), 0, 0)),
        ],
        out_shape=[
            jax.ShapeDtypeStruct((TBL_ROWS, ALIGN), jnp.float32),
            jax.ShapeDtypeStruct((B, 1, T), jnp.int32),
        ],
    )(x, W_lin, b_lin2, emb_speaker, spk3, dur3)


def _sc_body(tbl_h, dur_h, cse_h, f0_h, rmse_h, pos_h, wf0_h, bf0_h, wrm_h,
             brm_h, out_h, durv, csev, fidxv, f0v, rmv, posv, wf0v, bf0v,
             wrmv, brmv, buf, sem):
    cid = lax.axis_index("c")
    sid = lax.axis_index("s")
    wid = sid * NC + cid
    bb = wid // 2
    half = lax.rem(wid + bb, 2)
    base = bb * Y + half * (Y // 2)

    pltpu.sync_copy(dur_h.at[pl.ds(bb * T, T)], durv)
    pltpu.sync_copy(cse_h.at[pl.ds(bb * T, T)], csev)
    pltpu.sync_copy(f0_h.at[pl.ds(base, RPW)], f0v)
    pltpu.sync_copy(rmse_h.at[pl.ds(base, RPW)], rmv)
    pltpu.sync_copy(pos_h.at[pl.ds(base, RPW)], posv)
    pltpu.sync_copy(wf0_h, wf0v)
    pltpu.sync_copy(bf0_h, bf0v)
    pltpu.sync_copy(wrm_h, wrmv)
    pltpu.sync_copy(brm_h, brmv)

    zlane = jnp.full((16,), ZROW, jnp.int32) + lax.iota(jnp.int32, 16)

    def fill(i, c):
        fidxv[pl.ds(i * 16, 16)] = zlane + lax.rem(i * 16, T)
        return c

    lax.fori_loop(0, Y // 16, fill, 0)

    rowbase = bb * T
    for v in range(T // 16):
        dv = durv[pl.ds(v * 16, 16)]
        pos = csev[pl.ds(v * 16, 16)]
        vals = (jnp.full((16,), rowbase + v * 16, jnp.int32)
                + lax.iota(jnp.int32, 16))
        plsc.store_scatter(fidxv, [pos], vals, mask=dv >= 1)
        plsc.store_scatter(fidxv, [pos + 1], vals, mask=dv >= 2)
        plsc.store_scatter(fidxv, [pos + 2], vals, mask=dv >= 3)

    nj = FE // 16
    wf0r = [wf0v[pl.ds(j * 16, 16)] for j in range(nj)]
    bf0r = [bf0v[pl.ds(j * 16, 16)] for j in range(nj)]
    wrmr = [wrmv[pl.ds(j * 16, 16)] for j in range(nj)]
    brmr = [brmv[pl.ds(j * 16, 16)] for j in range(nj)]
    lane = lax.iota(jnp.int32, 16)
    poscol = jnp.full((16,), ALIGN + 2 * FE, jnp.int32)

    foff = half * (Y // 2)
    dtail = durv[pl.ds(T - 16, 16)]
    ctail = csev[pl.ds(T - 16, 16)]
    total = dtail[15] + ctail[15]
    zvec = jnp.zeros((16,), jnp.float32)

    def chunk_fn(c, acc):
        start = foff + c * CH

        @pl.when(total > start)
        def _():
            idx_slice = fidxv.at[pl.ds(start, CH)]
            pltpu.async_copy(tbl_h.at[idx_slice], buf.at[:, pl.ds(0, ALIGN)],
                             sem).wait()

        @pl.when(jnp.logical_and(
            total <= start,
            jnp.logical_or(c == 0, total > start - CH)))
        def _():
            def zrow_fn(r, acc2):
                for j in range(ALIGN // 16):
                    buf[r, pl.ds(j * 16, 16)] = zvec
                return acc2

            lax.fori_loop(0, CH, zrow_fn, 0)

        def grp_fn(g, acc2):
            rbase = c * CH + g * 16
            f0g = f0v[pl.ds(rbase, 16)]
            rmg = rmv[pl.ds(rbase, 16)]
            pog = posv[pl.ds(rbase, 16)]
            plsc.store_scatter(buf, [g * 16 + lane, poscol], pog)
            for r in range(16):
                row = g * 16 + r
                f0r = f0g[r]
                rmr = rmg[r]
                for j in range(nj):
                    buf[row, pl.ds(ALIGN + j * 16, 16)] = (
                        f0r * wf0r[j] + bf0r[j])
                    buf[row, pl.ds(ALIGN + FE + j * 16, 16)] = (
                        rmr * wrmr[j] + brmr[j])
            return acc2

        lax.fori_loop(0, CH // 16, grp_fn, 0)
        pltpu.sync_copy(buf, out_h.at[bb, pl.ds(start, CH)])
        return acc

    lax.fori_loop(0, NCH, chunk_fn, 0)


def _sc_expand(tbl, dur_flat, cse_flat, f0_flat, rmse_flat, pos_flat,
               wf0, bf0, wrm, brm):
    mesh = plsc.VectorSubcoreMesh(core_axis_name="c", subcore_axis_name="s")
    fn = functools.partial(
        pl.kernel,
        mesh=mesh,
        compiler_params=pltpu.CompilerParams(needs_layout_passes=False),
        out_type=jax.ShapeDtypeStruct((B, Y, OUT_W), jnp.float32),
        scratch_types=[
            pltpu.VMEM((T,), jnp.int32),
            pltpu.VMEM((T,), jnp.int32),
            pltpu.VMEM((Y,), jnp.int32),
            pltpu.VMEM((RPW,), jnp.float32),
            pltpu.VMEM((RPW,), jnp.float32),
            pltpu.VMEM((RPW,), jnp.float32),
            pltpu.VMEM((FE,), jnp.float32),
            pltpu.VMEM((FE,), jnp.float32),
            pltpu.VMEM((FE,), jnp.float32),
            pltpu.VMEM((FE,), jnp.float32),
            pltpu.VMEM((CH, OUT_W), jnp.float32),
            pltpu.SemaphoreType.DMA,
        ],
    )(_sc_body)
    return fn(tbl, dur_flat, cse_flat, f0_flat, rmse_flat, pos_flat,
              wf0, bf0, wrm, brm)


def kernel(x, f0, rmse, position, emb_speaker, W_f0, b_f0, W_rmse, b_rmse,
           W_lin, b_lin, speaker, duration, max_y_len):
    del max_y_len
    tbl, cse = _tc_table(x, W_lin, b_lin.reshape(1, ALIGN), emb_speaker,
                         speaker.reshape(B, 1, T), duration.reshape(B, 1, T))
    out = _sc_expand(tbl,
                     duration.reshape(-1), cse.reshape(-1),
                     f0.reshape(-1), rmse.reshape(-1), position.reshape(-1),
                     W_f0.reshape(-1), b_f0,
                     W_rmse.reshape(-1), b_rmse)
    return out
